# baseline (speedup 1.0000x reference)
import jax
import jax.numpy as jnp
from jax.experimental import pallas as pl
from jax.experimental.pallas import tpu as pltpu

_B, _T = 128, 1024
_CIN = 15
_EMB = 8
_CH = 64
_CPAD = 32
_DILS = (1, 2, 4, 8, 16)
_BB = 8
_BF = jnp.bfloat16


def _tcn_kernel(x_ref, w0a_ref, w0b_ref, w0r_ref, wa_ref, wb_ref,
                wh_ref, aux_ref, scal_ref, pet_ref, pck_ref, aet_ref,
                cwd_ref):
    aux = aux_ref[...]
    scal = scal_ref[...]

    def shift(vb, s):
        vi = pltpu.bitcast(vb, jnp.int32)
        sh = jnp.concatenate(
            [jnp.zeros((vi.shape[0], s), jnp.int32), vi[:, :_T - s]], axis=1)
        return pltpu.bitcast(sh, _BF)

    def conv3(vb, wcat, bias, d):
        xcat = jnp.concatenate([shift(vb, 2 * d), shift(vb, d), vb], axis=0)
        y = jnp.dot(wcat, xcat, preferred_element_type=jnp.float32)
        return jnp.maximum(y.astype(_BF) + bias, _BF(0.0))

    auxb = aux.astype(_BF)
    bb = range(_BB)
    xin = [x_ref[b] for b in bb]

    h = [conv3(xin[b], w0a_ref[...], auxb[:, 0:1], 1) for b in bb]
    h = [conv3(h[b], w0b_ref[...], auxb[:, 1:2], 1) for b in bb]
    res = [jnp.dot(w0r_ref[...], xin[b], preferred_element_type=jnp.float32)
           for b in bb]
    f = [h[b] + (res[b] + aux[:, 2:3]).astype(_BF) for b in bb]

    for i, d in enumerate(_DILS[1:]):
        h = [conv3(f[b], wa_ref[i], auxb[:, 3 + i:4 + i], d) for b in bb]
        h = [conv3(h[b], wb_ref[i], auxb[:, 7 + i:8 + i], d) for b in bb]
        f = [f[b] + h[b] for b in bb]

    g = [jnp.dot(wh_ref[...], f[b], preferred_element_type=jnp.float32)
         for b in bb]
    for b in bb:
        pet = jax.nn.softplus(g[b][0:1] + scal[0:1, 0:1])
        pck = jax.nn.softplus(g[b][1:2] + scal[0:1, 1:2])
        aet_lin = (g[b][2:3] + scal[0:1, 2:3]
                   + scal[0:1, 3:4] * pet + scal[0:1, 4:5] * pck)
        aet = jax.nn.sigmoid(aet_lin) * pet
        pet_ref[b, 0:1, :] = pet
        pck_ref[b, 0:1, :] = pck
        aet_ref[b, 0:1, :] = aet
        cwd_ref[b, 0:1, :] = pet - aet


def kernel(x, fveg_ids, fveg_emb, w0a, b0a, w0b, b0b, w0r, b0r,
           wa, ba, wb, bb, pet_w, pet_b, pck_w, pck_b, aet_w, aet_b):
    Bx, cin, Tt = x.shape
    nb = wa.shape[0]
    emb = fveg_emb.shape[1]

    fv = fveg_emb[fveg_ids]
    xinb = jnp.concatenate(
        [jnp.broadcast_to(fv[:, :, None], (Bx, emb, Tt)), x,
         jnp.zeros((Bx, _CPAD - emb - cin, Tt), jnp.float32)],
        axis=1).astype(_BF)

    def reorder(w):
        return jnp.concatenate(
            [w[:, cin:cin + emb], w[:, :cin],
             jnp.zeros((w.shape[0], _CPAD - cin - emb, w.shape[2]), w.dtype)],
            axis=1)

    def cat_taps(w):
        return w.transpose(0, 2, 1).reshape(w.shape[0], 3 * w.shape[1])

    w0a_c = cat_taps(reorder(w0a)).astype(_BF)
    w0b_c = cat_taps(w0b).astype(_BF)
    w0r_c = reorder(w0r)[:, :, 0].astype(_BF)
    wa_c = wa.transpose(0, 1, 3, 2).reshape(nb, _CH, 3 * _CH).astype(_BF)
    wb_c = wb.transpose(0, 1, 3, 2).reshape(nb, _CH, 3 * _CH).astype(_BF)
    wh = jnp.concatenate([pet_w[:, :, 0], pck_w[:, :, 0],
                          aet_w[:, :_CH, 0],
                          jnp.zeros((5, _CH), x.dtype)], axis=0).astype(_BF)

    aux = jnp.concatenate(
        [b0a[:, None], b0b[:, None], b0r[:, None], ba.T, bb.T], axis=1)
    scal = jnp.concatenate(
        [pet_b, pck_b, aet_b, aet_w[0, _CH:_CH + 2, 0],
         jnp.zeros((123,), jnp.float32)])[None, :]

    grid = (Bx // _BB,)
    out_sds = jax.ShapeDtypeStruct((Bx, 1, Tt), jnp.float32)
    out_spec = pl.BlockSpec((_BB, 1, Tt), lambda i: (i, 0, 0))
    pet, pck, aet, cwd = pl.pallas_call(
        _tcn_kernel,
        grid=grid,
        in_specs=[
            pl.BlockSpec((_BB, _CPAD, Tt), lambda i: (i, 0, 0)),
            pl.BlockSpec((_CH, 3 * _CPAD), lambda i: (0, 0)),
            pl.BlockSpec((_CH, 3 * _CH), lambda i: (0, 0)),
            pl.BlockSpec((_CH, _CPAD), lambda i: (0, 0)),
            pl.BlockSpec((nb, _CH, 3 * _CH), lambda i: (0, 0, 0)),
            pl.BlockSpec((nb, _CH, 3 * _CH), lambda i: (0, 0, 0)),
            pl.BlockSpec((8, _CH), lambda i: (0, 0)),
            pl.BlockSpec((_CH, 11), lambda i: (0, 0)),
            pl.BlockSpec((1, 128), lambda i: (0, 0)),
        ],
        out_specs=[out_spec, out_spec, out_spec, out_spec],
        out_shape=[out_sds, out_sds, out_sds, out_sds],
        compiler_params=pltpu.CompilerParams(
            dimension_semantics=("arbitrary",),
            vmem_limit_bytes=56 * 1024 * 1024,
        ),
    )(xinb, w0a_c, w0b_c, w0r_c, wa_c, wb_c, wh, aux, scal)

    return (pet, pck, aet, cwd)


# speedup vs baseline: 1.0598x; 1.0598x over previous
import jax
import jax.numpy as jnp
from jax.experimental import pallas as pl
from jax.experimental.pallas import tpu as pltpu

_B, _T = 128, 1024
_CIN = 15
_EMB = 8
_CH = 64
_CPAD = 32
_DILS = (1, 2, 4, 8, 16)
_BB = 16
_BF = jnp.bfloat16


def _tcn_kernel(x_ref, w0a_ref, w0b_ref, w0r_ref, wa_ref, wb_ref,
                wh_ref, aux_ref, scal_ref, pet_ref, pck_ref, aet_ref,
                cwd_ref):
    aux = aux_ref[...]
    scal = scal_ref[...]

    def shift(vb, s):
        vi = pltpu.bitcast(vb, jnp.int32)
        sh = jnp.concatenate(
            [jnp.zeros((vi.shape[0], s), jnp.int32), vi[:, :_T - s]], axis=1)
        return pltpu.bitcast(sh, _BF)

    def conv3(vb, wcat, bias, d):
        xcat = jnp.concatenate([shift(vb, 2 * d), shift(vb, d), vb], axis=0)
        y = jnp.dot(wcat, xcat, preferred_element_type=jnp.float32)
        return jnp.maximum(y.astype(_BF) + bias, _BF(0.0))

    auxb = aux.astype(_BF)
    bb = range(_BB)
    xin = [x_ref[b] for b in bb]

    h = [conv3(xin[b], w0a_ref[...], auxb[:, 0:1], 1) for b in bb]
    h = [conv3(h[b], w0b_ref[...], auxb[:, 1:2], 1) for b in bb]
    res = [jnp.dot(w0r_ref[...], xin[b], preferred_element_type=jnp.float32)
           for b in bb]
    f = [h[b] + (res[b] + aux[:, 2:3]).astype(_BF) for b in bb]

    for i, d in enumerate(_DILS[1:]):
        h = [conv3(f[b], wa_ref[i], auxb[:, 3 + i:4 + i], d) for b in bb]
        h = [conv3(h[b], wb_ref[i], auxb[:, 7 + i:8 + i], d) for b in bb]
        f = [f[b] + h[b] for b in bb]

    g = [jnp.dot(wh_ref[...], f[b], preferred_element_type=jnp.float32)
         for b in bb]
    for b in bb:
        pet = jax.nn.softplus(g[b][0:1] + scal[0:1, 0:1])
        pck = jax.nn.softplus(g[b][1:2] + scal[0:1, 1:2])
        aet_lin = (g[b][2:3] + scal[0:1, 2:3]
                   + scal[0:1, 3:4] * pet + scal[0:1, 4:5] * pck)
        aet = jax.nn.sigmoid(aet_lin) * pet
        pet_ref[b, 0:1, :] = pet
        pck_ref[b, 0:1, :] = pck
        aet_ref[b, 0:1, :] = aet
        cwd_ref[b, 0:1, :] = pet - aet


def kernel(x, fveg_ids, fveg_emb, w0a, b0a, w0b, b0b, w0r, b0r,
           wa, ba, wb, bb, pet_w, pet_b, pck_w, pck_b, aet_w, aet_b):
    Bx, cin, Tt = x.shape
    nb = wa.shape[0]
    emb = fveg_emb.shape[1]

    fv = fveg_emb[fveg_ids]
    xinb = jnp.concatenate(
        [jnp.broadcast_to(fv[:, :, None], (Bx, emb, Tt)), x,
         jnp.zeros((Bx, _CPAD - emb - cin, Tt), jnp.float32)],
        axis=1).astype(_BF)

    def reorder(w):
        return jnp.concatenate(
            [w[:, cin:cin + emb], w[:, :cin],
             jnp.zeros((w.shape[0], _CPAD - cin - emb, w.shape[2]), w.dtype)],
            axis=1)

    def cat_taps(w):
        return w.transpose(0, 2, 1).reshape(w.shape[0], 3 * w.shape[1])

    w0a_c = cat_taps(reorder(w0a)).astype(_BF)
    w0b_c = cat_taps(w0b).astype(_BF)
    w0r_c = reorder(w0r)[:, :, 0].astype(_BF)
    wa_c = wa.transpose(0, 1, 3, 2).reshape(nb, _CH, 3 * _CH).astype(_BF)
    wb_c = wb.transpose(0, 1, 3, 2).reshape(nb, _CH, 3 * _CH).astype(_BF)
    wh = jnp.concatenate([pet_w[:, :, 0], pck_w[:, :, 0],
                          aet_w[:, :_CH, 0],
                          jnp.zeros((5, _CH), x.dtype)], axis=0).astype(_BF)

    aux = jnp.concatenate(
        [b0a[:, None], b0b[:, None], b0r[:, None], ba.T, bb.T], axis=1)
    scal = jnp.concatenate(
        [pet_b, pck_b, aet_b, aet_w[0, _CH:_CH + 2, 0],
         jnp.zeros((123,), jnp.float32)])[None, :]

    grid = (Bx // _BB,)
    out_sds = jax.ShapeDtypeStruct((Bx, 1, Tt), jnp.float32)
    out_spec = pl.BlockSpec((_BB, 1, Tt), lambda i: (i, 0, 0))
    pet, pck, aet, cwd = pl.pallas_call(
        _tcn_kernel,
        grid=grid,
        in_specs=[
            pl.BlockSpec((_BB, _CPAD, Tt), lambda i: (i, 0, 0)),
            pl.BlockSpec((_CH, 3 * _CPAD), lambda i: (0, 0)),
            pl.BlockSpec((_CH, 3 * _CH), lambda i: (0, 0)),
            pl.BlockSpec((_CH, _CPAD), lambda i: (0, 0)),
            pl.BlockSpec((nb, _CH, 3 * _CH), lambda i: (0, 0, 0)),
            pl.BlockSpec((nb, _CH, 3 * _CH), lambda i: (0, 0, 0)),
            pl.BlockSpec((8, _CH), lambda i: (0, 0)),
            pl.BlockSpec((_CH, 11), lambda i: (0, 0)),
            pl.BlockSpec((1, 128), lambda i: (0, 0)),
        ],
        out_specs=[out_spec, out_spec, out_spec, out_spec],
        out_shape=[out_sds, out_sds, out_sds, out_sds],
        compiler_params=pltpu.CompilerParams(
            dimension_semantics=("arbitrary",),
            allow_input_fusion=(True, False, False, False, False, False, False, False, False),
            vmem_limit_bytes=56 * 1024 * 1024,
        ),
    )(xinb, w0a_c, w0b_c, w0r_c, wa_c, wb_c, wh, aux, scal)

    return (pet, pck, aet, cwd)


# speedup vs baseline: 1.0658x; 1.0056x over previous
import jax
import jax.numpy as jnp
from jax.experimental import pallas as pl
from jax.experimental.pallas import tpu as pltpu

_B, _T = 128, 1024
_CIN = 15
_EMB = 8
_CH = 64
_CPAD = 32
_DILS = (1, 2, 4, 8, 16)
_BB = 16
_BF = jnp.bfloat16


def _tcn_kernel(x_ref, w0a_ref, w0b_ref, w0r_ref, wa_ref, wb_ref,
                wh_ref, aux_ref, scal_ref, pet_ref, pck_ref, aet_ref,
                cwd_ref):
    aux = aux_ref[...]
    scal = scal_ref[...]

    def shift(vb, s):
        vi = pltpu.bitcast(vb, jnp.int32)
        sh = jnp.concatenate(
            [jnp.zeros((vi.shape[0], s), jnp.int32), vi[:, :_T - s]], axis=1)
        return pltpu.bitcast(sh, _BF)

    def conv3(vb, wcat, bias, d):
        xcat = jnp.concatenate([shift(vb, 2 * d), shift(vb, d), vb], axis=0)
        y = jnp.dot(wcat, xcat, preferred_element_type=jnp.float32)
        return jnp.maximum(y.astype(_BF) + bias, _BF(0.0))

    auxb = aux.astype(_BF)
    bb = range(_BB)
    xin = [x_ref[b] for b in bb]

    h = [conv3(xin[b], w0a_ref[...], auxb[:, 0:1], 1) for b in bb]
    h = [conv3(h[b], w0b_ref[...], auxb[:, 1:2], 1) for b in bb]
    res = [jnp.dot(w0r_ref[...], xin[b], preferred_element_type=jnp.float32)
           for b in bb]
    f = [h[b] + (res[b] + aux[:, 2:3]).astype(_BF) for b in bb]

    for i, d in enumerate(_DILS[1:]):
        h = [conv3(f[b], wa_ref[i], auxb[:, 3 + i:4 + i], d) for b in bb]
        h = [conv3(h[b], wb_ref[i], auxb[:, 7 + i:8 + i], d) for b in bb]
        f = [f[b] + h[b] for b in bb]

    g = [jnp.dot(wh_ref[...], f[b], preferred_element_type=jnp.float32)
         for b in bb]
    for b in bb:
        pet = jax.nn.softplus(g[b][0:1] + scal[0:1, 0:1])
        pck = jax.nn.softplus(g[b][1:2] + scal[0:1, 1:2])
        aet_lin = (g[b][2:3] + scal[0:1, 2:3]
                   + scal[0:1, 3:4] * pet + scal[0:1, 4:5] * pck)
        aet = jax.nn.sigmoid(aet_lin) * pet
        pet_ref[b, 0:1, :] = pet
        pck_ref[b, 0:1, :] = pck
        aet_ref[b, 0:1, :] = aet
        cwd_ref[b, 0:1, :] = pet - aet


def kernel(x, fveg_ids, fveg_emb, w0a, b0a, w0b, b0b, w0r, b0r,
           wa, ba, wb, bb, pet_w, pet_b, pck_w, pck_b, aet_w, aet_b):
    Bx, cin, Tt = x.shape
    nb = wa.shape[0]
    emb = fveg_emb.shape[1]

    fv = fveg_emb[fveg_ids]
    xinb = jnp.concatenate(
        [jnp.broadcast_to(fv[:, :, None], (Bx, emb, Tt)), x,
         jnp.zeros((Bx, _CPAD - emb - cin, Tt), jnp.float32)],
        axis=1).astype(_BF)

    def reorder(w):
        return jnp.concatenate(
            [w[:, cin:cin + emb], w[:, :cin],
             jnp.zeros((w.shape[0], _CPAD - cin - emb, w.shape[2]), w.dtype)],
            axis=1)

    def cat_taps(w):
        return w.transpose(0, 2, 1).reshape(w.shape[0], 3 * w.shape[1])

    w0a_c = cat_taps(reorder(w0a)).astype(_BF)
    w0b_c = cat_taps(w0b).astype(_BF)
    w0r_c = reorder(w0r)[:, :, 0].astype(_BF)
    wa_c = wa.transpose(0, 1, 3, 2).reshape(nb, _CH, 3 * _CH).astype(_BF)
    wb_c = wb.transpose(0, 1, 3, 2).reshape(nb, _CH, 3 * _CH).astype(_BF)
    wh = jnp.concatenate([pet_w[:, :, 0], pck_w[:, :, 0],
                          aet_w[:, :_CH, 0],
                          jnp.zeros((5, _CH), x.dtype)], axis=0).astype(_BF)

    aux = jnp.concatenate(
        [b0a[:, None], b0b[:, None], b0r[:, None], ba.T, bb.T], axis=1)
    scal = jnp.concatenate(
        [pet_b, pck_b, aet_b, aet_w[0, _CH:_CH + 2, 0],
         jnp.zeros((123,), jnp.float32)])[None, :]

    grid = (Bx // _BB,)
    out_sds = jax.ShapeDtypeStruct((Bx, 1, Tt), jnp.float32)
    out_spec = pl.BlockSpec((_BB, 1, Tt), lambda i: (i, 0, 0))
    pet, pck, aet, cwd = pl.pallas_call(
        _tcn_kernel,
        grid=grid,
        in_specs=[
            pl.BlockSpec((_BB, _CPAD, Tt), lambda i: (i, 0, 0)),
            pl.BlockSpec((_CH, 3 * _CPAD), lambda i: (0, 0)),
            pl.BlockSpec((_CH, 3 * _CH), lambda i: (0, 0)),
            pl.BlockSpec((_CH, _CPAD), lambda i: (0, 0)),
            pl.BlockSpec((nb, _CH, 3 * _CH), lambda i: (0, 0, 0)),
            pl.BlockSpec((nb, _CH, 3 * _CH), lambda i: (0, 0, 0)),
            pl.BlockSpec((8, _CH), lambda i: (0, 0)),
            pl.BlockSpec((_CH, 11), lambda i: (0, 0)),
            pl.BlockSpec((1, 128), lambda i: (0, 0)),
        ],
        out_specs=[out_spec, out_spec, out_spec, out_spec],
        out_shape=[out_sds, out_sds, out_sds, out_sds],
        compiler_params=pltpu.CompilerParams(
            dimension_semantics=("arbitrary",),
            allow_input_fusion=(True, False, False, False, False, False, False, False, False),
            vmem_limit_bytes=56 * 1024 * 1024,
            flags={"XLA_TPU_STORE_TO_LOAD_FORWARDING_WINDOW": 12288},
        ),
    )(xinb, w0a_c, w0b_c, w0r_c, wa_c, wb_c, wh, aux, scal)

    return (pet, pck, aet, cwd)
